# Initial kernel scaffold; baseline (speedup 1.0000x reference)
import functools

import jax
import jax.numpy as jnp
import numpy as np
from jax import lax
from jax.experimental import pallas as pl
from jax.experimental.pallas import tpu as pltpu
from jax.experimental.pallas import tpu_sc as plsc

N = 10000
E = 320000
D = 128
H = 8
DK = 16
ROW = 144

NC = 2
NS = 16
NW = NC * NS
EPW = E // NW
CH = 80
NCHUNK = EPW // CH

_mesh = plsc.VectorSubcoreMesh(core_axis_name="c", subcore_axis_name="s")


def _head_selector():
    d = lax.broadcasted_iota(jnp.int32, (D, H), 0)
    h = lax.broadcasted_iota(jnp.int32, (D, H), 1)
    return (d // DK == h).astype(jnp.float32)


def _proj_body(x_ref, wa_ref, ba_ref, wc_ref, bc_ref,
               h_ref, q_ref, k_ref, v_ref):
    h = jax.nn.gelu(
        jnp.dot(x_ref[...], wa_ref[...], preferred_element_type=jnp.float32)
        + ba_ref[...], approximate=False)
    h_ref[...] = h
    cat = jnp.dot(h, wc_ref[...], preferred_element_type=jnp.float32) + bc_ref[...]
    q_ref[...] = cat[:, 0:D]
    k_ref[...] = cat[:, D:2 * D]
    v_ref[...] = cat[:, 2 * D:3 * D]


def _phase_a(x, W_adapt, b_adapt, Wcat, bcat):
    BN = 1000
    grid = (N // BN,)
    return pl.pallas_call(
        _proj_body,
        grid=grid,
        in_specs=[
            pl.BlockSpec((BN, D), lambda i: (i, 0)),
            pl.BlockSpec((D, D), lambda i: (0, 0)),
            pl.BlockSpec((1, D), lambda i: (0, 0)),
            pl.BlockSpec((D, 3 * D), lambda i: (0, 0)),
            pl.BlockSpec((1, 3 * D), lambda i: (0, 0)),
        ],
        out_specs=[pl.BlockSpec((BN, D), lambda i: (i, 0))] * 4,
        out_shape=[jax.ShapeDtypeStruct((N, D), jnp.float32)] * 4,
    )(x, W_adapt, b_adapt, Wcat, bcat)


@functools.partial(
    pl.kernel,
    out_type=[jax.ShapeDtypeStruct((E, D), jnp.float32)] * 3,
    mesh=_mesh,
    scratch_types=[
        pltpu.VMEM((CH,), jnp.int32),
        pltpu.VMEM((CH,), jnp.int32),
        pltpu.VMEM((CH, D), jnp.float32),
        pltpu.VMEM((CH, D), jnp.float32),
        pltpu.VMEM((CH, D), jnp.float32),
        pltpu.SemaphoreType.DMA,
    ],
)
def _gather_kernel(q_hbm, k_hbm, v_hbm, dst_hbm, src_hbm,
                   qd_out, ks_out, vs_out,
                   di_v, si_v, qb, kb, vb, sem):
    wid = lax.axis_index("s") * NC + lax.axis_index("c")
    base0 = wid * EPW

    def chunk(i, carry):
        base = base0 + i * CH
        pltpu.sync_copy(dst_hbm.at[pl.ds(base, CH)], di_v)
        pltpu.sync_copy(src_hbm.at[pl.ds(base, CH)], si_v)
        pltpu.async_copy(q_hbm.at[di_v], qb, sem).wait()
        pltpu.async_copy(k_hbm.at[si_v], kb, sem).wait()
        pltpu.async_copy(v_hbm.at[si_v], vb, sem).wait()
        pltpu.sync_copy(qb, qd_out.at[pl.ds(base, CH)])
        pltpu.sync_copy(kb, ks_out.at[pl.ds(base, CH)])
        pltpu.sync_copy(vb, vs_out.at[pl.ds(base, CH)])
        return carry

    lax.fori_loop(0, NCHUNK, chunk, 0)


def _edge_body(qd_ref, ks_ref, vs_ref, out_ref):
    S = _head_selector()
    p = qd_ref[...] * ks_ref[...]
    t = jnp.dot(p, S, preferred_element_type=jnp.float32)
    ex = jnp.exp(t)
    msg = vs_ref[...] * jnp.dot(ex, S.T, preferred_element_type=jnp.float32)
    out_ref[:, 0:D] = msg
    out_ref[:, D:ROW] = jnp.concatenate(
        [ex, jnp.zeros((ex.shape[0], ROW - D - H), jnp.float32)], axis=1)


def _phase_c(qd, ks, vs):
    BE = 2000
    grid = (E // BE,)
    return pl.pallas_call(
        _edge_body,
        grid=grid,
        in_specs=[pl.BlockSpec((BE, D), lambda i: (i, 0))] * 3,
        out_specs=pl.BlockSpec((BE, ROW), lambda i: (i, 0)),
        out_shape=jax.ShapeDtypeStruct((E, ROW), jnp.float32),
    )(qd, ks, vs)


@functools.partial(
    pl.kernel,
    out_type=jax.ShapeDtypeStruct((NC, N, ROW), jnp.float32),
    mesh=_mesh,
    scratch_types=[
        pltpu.VMEM((CH,), jnp.int32),
        pltpu.VMEM((CH, ROW), jnp.float32),
        pltpu.VMEM_SHARED((N, ROW), jnp.float32),
    ],
)
def _scatter_kernel(msg_hbm, dst_hbm, zeros_hbm, out_hbm, di_v, mb, table):
    cid = lax.axis_index("c")
    sid = lax.axis_index("s")
    wid = sid * NC + cid
    base0 = wid * EPW

    @pl.when(sid == 0)
    def _init():
        pltpu.sync_copy(zeros_hbm, table)

    plsc.subcore_barrier()

    def chunk(i, carry):
        base = base0 + i * CH
        pltpu.sync_copy(dst_hbm.at[pl.ds(base, CH)], di_v)
        pltpu.sync_copy(msg_hbm.at[pl.ds(base, CH)], mb)
        pltpu.sync_copy(mb, table.at[di_v], add=True)
        return carry

    lax.fori_loop(0, NCHUNK, chunk, 0)

    plsc.subcore_barrier()

    @pl.when(sid == 0)
    def _dump():
        pltpu.sync_copy(table, out_hbm.at[cid])


def _epi_body(slab_ref, h_ref, wa_ref, ba_ref, al_ref, g_ref, b_ref,
              wo_ref, bo_ref, o_ref):
    S = _head_selector()
    s = slab_ref[0] + slab_ref[1]
    num = s[:, 0:D]
    den = s[:, D:D + H]
    agg = num / (jnp.dot(den, S.T, preferred_element_type=jnp.float32) + 1e-16)
    out = jnp.dot(agg, wa_ref[...], preferred_element_type=jnp.float32) + ba_ref[...]
    alpha = al_ref[0, 0]
    out = out * alpha + h_ref[...] * (1.0 - alpha)
    mu = jnp.mean(out, axis=1, keepdims=True)
    var = jnp.mean((out - mu) ** 2, axis=1, keepdims=True)
    out = (out - mu) / jnp.sqrt(var + 1e-5) * g_ref[...] + b_ref[...]
    o_ref[...] = jnp.dot(out, wo_ref[...], preferred_element_type=jnp.float32) + bo_ref[...]


def _phase_e(slab, h, Wa, ba, alpha, ln_g, ln_b, W_out, b_out):
    BN = 1000
    grid = (N // BN,)
    return pl.pallas_call(
        _epi_body,
        grid=grid,
        in_specs=[
            pl.BlockSpec((NC, BN, ROW), lambda i: (0, i, 0)),
            pl.BlockSpec((BN, D), lambda i: (i, 0)),
            pl.BlockSpec((D, D), lambda i: (0, 0)),
            pl.BlockSpec((1, D), lambda i: (0, 0)),
            pl.BlockSpec((1, 1), lambda i: (0, 0)),
            pl.BlockSpec((1, D), lambda i: (0, 0)),
            pl.BlockSpec((1, D), lambda i: (0, 0)),
            pl.BlockSpec((D, D), lambda i: (0, 0)),
            pl.BlockSpec((1, D), lambda i: (0, 0)),
        ],
        out_specs=pl.BlockSpec((BN, D), lambda i: (i, 0)),
        out_shape=jax.ShapeDtypeStruct((N, D), jnp.float32),
    )(slab, h, Wa, ba, alpha, ln_g, ln_b, W_out, b_out)


def kernel(x, edge_index, W_adapt, b_adapt, Wk, bk, Wv, bv, Wq, bq,
           Wa, ba, rel_pri, rel_att, rel_msg, skip, ln_g, ln_b,
           W_out, b_out):
    f32 = jnp.float32
    scale = jnp.repeat(rel_pri[0] / np.sqrt(DK), DK)
    Wq_eff = Wq * scale[None, :]
    bq_eff = bq * scale
    Wk_eff = jnp.einsum('dhi,hij->dhj', Wk.reshape(D, H, DK), rel_att[0]).reshape(D, D)
    bk_eff = jnp.einsum('hi,hij->hj', bk.reshape(H, DK), rel_att[0]).reshape(D)
    Wv_eff = jnp.einsum('dhi,hij->dhj', Wv.reshape(D, H, DK), rel_msg[0]).reshape(D, D)
    bv_eff = jnp.einsum('hi,hij->hj', bv.reshape(H, DK), rel_msg[0]).reshape(D)
    Wcat = jnp.concatenate([Wq_eff, Wk_eff, Wv_eff], axis=1)
    bcat = jnp.concatenate([bq_eff, bk_eff, bv_eff])[None, :]

    h, q, k, v = _phase_a(x.astype(f32), W_adapt, b_adapt[None, :], Wcat, bcat)

    src = edge_index[0].astype(jnp.int32)
    dst = edge_index[1].astype(jnp.int32)

    qd, ks, vs = _gather_kernel(q, k, v, dst, src)
    m144 = _phase_c(qd, ks, vs)
    slab = _scatter_kernel(m144, dst, jnp.zeros((N, ROW), f32))

    alpha = jax.nn.sigmoid(skip[0]).reshape(1, 1)
    return _phase_e(slab, h, Wa, ba[None, :], alpha,
                    ln_g[None, :], ln_b[None, :], W_out, b_out)


# speedup vs baseline: 27.1550x; 27.1550x over previous
import functools

import jax
import jax.numpy as jnp
import numpy as np
from jax import lax
from jax.experimental import pallas as pl
from jax.experimental.pallas import tpu as pltpu
from jax.experimental.pallas import tpu_sc as plsc

N = 10000
E = 320000
D = 128
H = 8
DK = 16
ROW = 144

NC = 2
NS = 16
NW = NC * NS
EPW = E // NW
CH = 80
NCHUNK = EPW // CH

_mesh = plsc.VectorSubcoreMesh(core_axis_name="c", subcore_axis_name="s")


def _head_selector():
    d = lax.broadcasted_iota(jnp.int32, (D, H), 0)
    h = lax.broadcasted_iota(jnp.int32, (D, H), 1)
    return (d // DK == h).astype(jnp.float32)


def _proj_body(x_ref, wa_ref, ba_ref, wc_ref, bc_ref,
               h_ref, q_ref, k_ref, v_ref):
    u = (jnp.dot(x_ref[...], wa_ref[...], preferred_element_type=jnp.float32)
         + ba_ref[...])
    h = 0.5 * u * (1.0 + lax.erf(u * np.float32(1.0 / np.sqrt(2.0))))
    h_ref[...] = h
    cat = jnp.dot(h, wc_ref[...], preferred_element_type=jnp.float32) + bc_ref[...]
    q_ref[...] = cat[:, 0:D]
    k_ref[...] = cat[:, D:2 * D]
    v_ref[...] = cat[:, 2 * D:3 * D]


def _phase_a(x, W_adapt, b_adapt, Wcat, bcat):
    BN = 1000
    grid = (N // BN,)
    return pl.pallas_call(
        _proj_body,
        grid=grid,
        in_specs=[
            pl.BlockSpec((BN, D), lambda i: (i, 0)),
            pl.BlockSpec((D, D), lambda i: (0, 0)),
            pl.BlockSpec((1, D), lambda i: (0, 0)),
            pl.BlockSpec((D, 3 * D), lambda i: (0, 0)),
            pl.BlockSpec((1, 3 * D), lambda i: (0, 0)),
        ],
        out_specs=[pl.BlockSpec((BN, D), lambda i: (i, 0))] * 4,
        out_shape=[jax.ShapeDtypeStruct((N, D), jnp.float32)] * 4,
    )(x, W_adapt, b_adapt, Wcat, bcat)


@functools.partial(
    pl.kernel,
    out_type=[jax.ShapeDtypeStruct((E, D), jnp.float32)] * 3,
    mesh=_mesh,
    scratch_types=[
        pltpu.VMEM((CH,), jnp.int32),
        pltpu.VMEM((CH,), jnp.int32),
        pltpu.VMEM((CH, D), jnp.float32),
        pltpu.VMEM((CH, D), jnp.float32),
        pltpu.VMEM((CH, D), jnp.float32),
        pltpu.SemaphoreType.DMA,
    ],
)
def _gather_kernel(q_hbm, k_hbm, v_hbm, dst_hbm, src_hbm,
                   qd_out, ks_out, vs_out,
                   di_v, si_v, qb, kb, vb, sem):
    wid = lax.axis_index("s") * NC + lax.axis_index("c")
    base0 = wid * EPW

    def chunk(i, carry):
        base = base0 + i * CH
        pltpu.sync_copy(dst_hbm.at[pl.ds(base, CH)], di_v)
        pltpu.sync_copy(src_hbm.at[pl.ds(base, CH)], si_v)
        pltpu.async_copy(q_hbm.at[di_v], qb, sem).wait()
        pltpu.async_copy(k_hbm.at[si_v], kb, sem).wait()
        pltpu.async_copy(v_hbm.at[si_v], vb, sem).wait()
        pltpu.sync_copy(qb, qd_out.at[pl.ds(base, CH)])
        pltpu.sync_copy(kb, ks_out.at[pl.ds(base, CH)])
        pltpu.sync_copy(vb, vs_out.at[pl.ds(base, CH)])
        return carry

    lax.fori_loop(0, NCHUNK, chunk, 0)


def _edge_body(qd_ref, ks_ref, vs_ref, msg_ref, exr_ref):
    S = _head_selector()
    p = qd_ref[...] * ks_ref[...]
    t = jnp.dot(p, S, preferred_element_type=jnp.float32)
    ex = jnp.exp(t)
    exr = jnp.dot(ex, S.T, preferred_element_type=jnp.float32)
    msg_ref[...] = vs_ref[...] * exr
    exr_ref[...] = exr


def _phase_c(qd, ks, vs):
    BE = 2000
    grid = (E // BE,)
    return pl.pallas_call(
        _edge_body,
        grid=grid,
        in_specs=[pl.BlockSpec((BE, D), lambda i: (i, 0))] * 3,
        out_specs=[pl.BlockSpec((BE, D), lambda i: (i, 0))] * 2,
        out_shape=[jax.ShapeDtypeStruct((E, D), jnp.float32)] * 2,
    )(qd, ks, vs)


@functools.partial(
    pl.kernel,
    out_type=jax.ShapeDtypeStruct((NC, N, D), jnp.float32),
    mesh=_mesh,
    scratch_types=[
        pltpu.VMEM((CH,), jnp.int32),
        pltpu.VMEM((CH, D), jnp.float32),
        pltpu.VMEM_SHARED((N, D), jnp.float32),
    ],
)
def _scatter_kernel(msg_hbm, dst_hbm, zeros_hbm, out_hbm, di_v, mb, table):
    cid = lax.axis_index("c")
    sid = lax.axis_index("s")
    wid = sid * NC + cid
    base0 = wid * EPW

    @pl.when(sid == 0)
    def _init():
        pltpu.sync_copy(zeros_hbm, table)

    plsc.subcore_barrier()

    def chunk(i, carry):
        base = base0 + i * CH
        pltpu.sync_copy(dst_hbm.at[pl.ds(base, CH)], di_v)
        pltpu.sync_copy(msg_hbm.at[pl.ds(base, CH)], mb)
        pltpu.sync_copy(mb, table.at[di_v], add=True)
        return carry

    lax.fori_loop(0, NCHUNK, chunk, 0)

    plsc.subcore_barrier()

    @pl.when(sid == 0)
    def _dump():
        pltpu.sync_copy(table, out_hbm.at[cid])


def _epi_body(mslab_ref, dslab_ref, h_ref, wa_ref, ba_ref, al_ref, g_ref, b_ref,
              wo_ref, bo_ref, o_ref):
    num = mslab_ref[0] + mslab_ref[1]
    den = dslab_ref[0] + dslab_ref[1]
    agg = num / (den + 1e-16)
    out = jnp.dot(agg, wa_ref[...], preferred_element_type=jnp.float32) + ba_ref[...]
    alpha = al_ref[0, 0]
    out = out * alpha + h_ref[...] * (1.0 - alpha)
    mu = jnp.mean(out, axis=1, keepdims=True)
    var = jnp.mean((out - mu) ** 2, axis=1, keepdims=True)
    out = (out - mu) / jnp.sqrt(var + 1e-5) * g_ref[...] + b_ref[...]
    o_ref[...] = jnp.dot(out, wo_ref[...], preferred_element_type=jnp.float32) + bo_ref[...]


def _phase_e(mslab, dslab, h, Wa, ba, alpha, ln_g, ln_b, W_out, b_out):
    BN = 1000
    grid = (N // BN,)
    return pl.pallas_call(
        _epi_body,
        grid=grid,
        in_specs=[
            pl.BlockSpec((NC, BN, D), lambda i: (0, i, 0)),
            pl.BlockSpec((NC, BN, D), lambda i: (0, i, 0)),
            pl.BlockSpec((BN, D), lambda i: (i, 0)),
            pl.BlockSpec((D, D), lambda i: (0, 0)),
            pl.BlockSpec((1, D), lambda i: (0, 0)),
            pl.BlockSpec((1, 1), lambda i: (0, 0)),
            pl.BlockSpec((1, D), lambda i: (0, 0)),
            pl.BlockSpec((1, D), lambda i: (0, 0)),
            pl.BlockSpec((D, D), lambda i: (0, 0)),
            pl.BlockSpec((1, D), lambda i: (0, 0)),
        ],
        out_specs=pl.BlockSpec((BN, D), lambda i: (i, 0)),
        out_shape=jax.ShapeDtypeStruct((N, D), jnp.float32),
    )(mslab, dslab, h, Wa, ba, alpha, ln_g, ln_b, W_out, b_out)


def kernel(x, edge_index, W_adapt, b_adapt, Wk, bk, Wv, bv, Wq, bq,
           Wa, ba, rel_pri, rel_att, rel_msg, skip, ln_g, ln_b,
           W_out, b_out):
    f32 = jnp.float32
    scale = jnp.repeat(rel_pri[0] / np.sqrt(DK), DK)
    Wq_eff = Wq * scale[None, :]
    bq_eff = bq * scale
    Wk_eff = jnp.einsum('dhi,hij->dhj', Wk.reshape(D, H, DK), rel_att[0]).reshape(D, D)
    bk_eff = jnp.einsum('hi,hij->hj', bk.reshape(H, DK), rel_att[0]).reshape(D)
    Wv_eff = jnp.einsum('dhi,hij->dhj', Wv.reshape(D, H, DK), rel_msg[0]).reshape(D, D)
    bv_eff = jnp.einsum('hi,hij->hj', bv.reshape(H, DK), rel_msg[0]).reshape(D)
    Wcat = jnp.concatenate([Wq_eff, Wk_eff, Wv_eff], axis=1)
    bcat = jnp.concatenate([bq_eff, bk_eff, bv_eff])[None, :]

    h, q, k, v = _phase_a(x.astype(f32), W_adapt, b_adapt[None, :], Wcat, bcat)

    src = edge_index[0].astype(jnp.int32)
    dst = edge_index[1].astype(jnp.int32)

    qd, ks, vs = _gather_kernel(q, k, v, dst, src)
    msg, exr = _phase_c(qd, ks, vs)
    zeros = jnp.zeros((N, D), f32)
    mslab = _scatter_kernel(msg, dst, zeros)
    dslab = _scatter_kernel(exr, dst, zeros)

    alpha = jax.nn.sigmoid(skip[0]).reshape(1, 1)
    return _phase_e(mslab, dslab, h, Wa, ba[None, :], alpha,
                    ln_g[None, :], ln_b[None, :], W_out, b_out[None, :])


# speedup vs baseline: 42.8986x; 1.5798x over previous
import functools

import jax
import jax.numpy as jnp
import numpy as np
from jax import lax
from jax.experimental import pallas as pl
from jax.experimental.pallas import tpu as pltpu
from jax.experimental.pallas import tpu_sc as plsc

N = 10000
E = 320000
D = 128
H = 8
DK = 16
ROW = 144

NC = 2
NS = 16
NW = NC * NS
EPW = E // NW
CH = 128
NCH_G = EPW // CH
CH_GT = EPW - NCH_G * CH
EPT = E // NS
NCH_S = EPT // CH
CH_ST = EPT - NCH_S * CH

_mesh = plsc.VectorSubcoreMesh(core_axis_name="c", subcore_axis_name="s")


def _head_selector():
    d = lax.broadcasted_iota(jnp.int32, (D, H), 0)
    h = lax.broadcasted_iota(jnp.int32, (D, H), 1)
    return (d // DK == h).astype(jnp.float32)


def _proj_body(x_ref, wa_ref, ba_ref, wc_ref, bc_ref,
               h_ref, q_ref, k_ref, v_ref):
    u = (jnp.dot(x_ref[...], wa_ref[...], preferred_element_type=jnp.float32)
         + ba_ref[...])
    h = 0.5 * u * (1.0 + lax.erf(u * np.float32(1.0 / np.sqrt(2.0))))
    h_ref[...] = h
    cat = jnp.dot(h, wc_ref[...], preferred_element_type=jnp.float32) + bc_ref[...]
    q_ref[...] = cat[:, 0:D]
    k_ref[...] = cat[:, D:2 * D]
    v_ref[...] = cat[:, 2 * D:3 * D]


def _phase_a(x, W_adapt, b_adapt, Wcat, bcat):
    BN = 1000
    grid = (N // BN,)
    return pl.pallas_call(
        _proj_body,
        grid=grid,
        in_specs=[
            pl.BlockSpec((BN, D), lambda i: (i, 0)),
            pl.BlockSpec((D, D), lambda i: (0, 0)),
            pl.BlockSpec((1, D), lambda i: (0, 0)),
            pl.BlockSpec((D, 3 * D), lambda i: (0, 0)),
            pl.BlockSpec((1, 3 * D), lambda i: (0, 0)),
        ],
        out_specs=[pl.BlockSpec((BN, D), lambda i: (i, 0))] * 4,
        out_shape=[jax.ShapeDtypeStruct((N, D), jnp.float32)] * 4,
    )(x, W_adapt, b_adapt, Wcat, bcat)


@functools.partial(
    pl.kernel,
    out_type=[jax.ShapeDtypeStruct((E, D), jnp.float32)] * 3,
    mesh=_mesh,
    scratch_types=[
        pltpu.VMEM((2, CH), jnp.int32),
        pltpu.VMEM((2, CH), jnp.int32),
        pltpu.VMEM((2, CH, D), jnp.float32),
        pltpu.VMEM((2, CH, D), jnp.float32),
        pltpu.VMEM((2, CH, D), jnp.float32),
        pltpu.VMEM((CH_GT,), jnp.int32),
        pltpu.VMEM((CH_GT,), jnp.int32),
        pltpu.VMEM((CH_GT, D), jnp.float32),
        pltpu.VMEM((CH_GT, D), jnp.float32),
        pltpu.VMEM((CH_GT, D), jnp.float32),
        pltpu.SemaphoreType.DMA,
        pltpu.SemaphoreType.DMA,
        pltpu.SemaphoreType.DMA,
    ],
)
def _gather_kernel(q_hbm, k_hbm, v_hbm, dst_hbm, src_hbm,
                   qd_out, ks_out, vs_out,
                   di, si, qb, kb, vb, dit, sit, qbt, kbt, vbt,
                   gsem, wsem0, wsem1):
    wid = lax.axis_index("s") * NC + lax.axis_index("c")
    base0 = wid * EPW
    wsems = (wsem0, wsem1)

    def do_chunk(base, n, div, siv, qbv, kbv, vbv, wsem):
        pltpu.sync_copy(dst_hbm.at[pl.ds(base, n)], div)
        pltpu.sync_copy(src_hbm.at[pl.ds(base, n)], siv)
        c1 = pltpu.async_copy(q_hbm.at[div], qbv, gsem)
        c2 = pltpu.async_copy(k_hbm.at[siv], kbv, gsem)
        c3 = pltpu.async_copy(v_hbm.at[siv], vbv, gsem)
        c1.wait()
        c2.wait()
        c3.wait()
        pltpu.async_copy(qbv, qd_out.at[pl.ds(base, n)], wsem)
        pltpu.async_copy(kbv, ks_out.at[pl.ds(base, n)], wsem)
        pltpu.async_copy(vbv, vs_out.at[pl.ds(base, n)], wsem)

    def drain_writes(b, base, n, wsem):
        pltpu.make_async_copy(qb.at[b].at[pl.ds(0, n)], qd_out.at[pl.ds(base, n)], wsem).wait()
        pltpu.make_async_copy(kb.at[b].at[pl.ds(0, n)], ks_out.at[pl.ds(base, n)], wsem).wait()
        pltpu.make_async_copy(vb.at[b].at[pl.ds(0, n)], vs_out.at[pl.ds(base, n)], wsem).wait()

    def outer(j, carry):
        for b in range(2):
            i = j * 2 + b
            base = base0 + i * CH

            @pl.when(j > 0)
            def _():
                drain_writes(b, base, CH, wsems[b])

            do_chunk(base, CH, di.at[b], si.at[b], qb.at[b], kb.at[b], vb.at[b],
                     wsems[b])
        return carry

    lax.fori_loop(0, NCH_G // 2, outer, 0)
    for b in range(2):
        drain_writes(b, base0, CH, wsems[b])
    tbase = base0 + NCH_G * CH
    do_chunk(tbase, CH_GT, dit, sit, qbt, kbt, vbt, gsem)
    pltpu.make_async_copy(qbt, qd_out.at[pl.ds(tbase, CH_GT)], gsem).wait()
    pltpu.make_async_copy(kbt, ks_out.at[pl.ds(tbase, CH_GT)], gsem).wait()
    pltpu.make_async_copy(vbt, vs_out.at[pl.ds(tbase, CH_GT)], gsem).wait()


def _edge_body(qd_ref, ks_ref, vs_ref, msg_ref, exr_ref):
    S = _head_selector()
    p = qd_ref[...] * ks_ref[...]
    t = jnp.dot(p, S, preferred_element_type=jnp.float32)
    ex = jnp.exp(t)
    exr = jnp.dot(ex, S.T, preferred_element_type=jnp.float32)
    msg_ref[...] = vs_ref[...] * exr
    exr_ref[...] = exr


def _phase_c(qd, ks, vs):
    BE = 2000
    grid = (E // BE,)
    return pl.pallas_call(
        _edge_body,
        grid=grid,
        in_specs=[pl.BlockSpec((BE, D), lambda i: (i, 0))] * 3,
        out_specs=[pl.BlockSpec((BE, D), lambda i: (i, 0))] * 2,
        out_shape=[jax.ShapeDtypeStruct((E, D), jnp.float32)] * 2,
    )(qd, ks, vs)


@functools.partial(
    pl.kernel,
    out_type=jax.ShapeDtypeStruct((NC, N, D), jnp.float32),
    mesh=_mesh,
    scratch_types=[
        pltpu.VMEM((2, CH), jnp.int32),
        pltpu.VMEM((2, CH, D), jnp.float32),
        pltpu.VMEM((CH_ST,), jnp.int32),
        pltpu.VMEM((CH_ST, D), jnp.float32),
        pltpu.VMEM_SHARED((N, D), jnp.float32),
        pltpu.SemaphoreType.DMA,
    ],
)
def _scatter_kernel(msg_hbm, exr_hbm, dst_hbm, zeros_hbm, out_hbm,
                    di, mb, dit, mbt, table, lsem):
    cid = lax.axis_index("c")
    sid = lax.axis_index("s")
    base0 = sid * EPT

    @pl.when(sid == 0)
    def _init():
        pltpu.sync_copy(zeros_hbm, table)

    plsc.subcore_barrier()

    def run(rows_hbm):
        pltpu.sync_copy(dst_hbm.at[pl.ds(base0, CH)], di.at[0])
        pltpu.async_copy(rows_hbm.at[pl.ds(base0, CH)], mb.at[0], lsem)

        def outer(j, carry):
            for b in range(2):
                i = j * 2 + b
                base = base0 + i * CH
                pltpu.make_async_copy(rows_hbm.at[pl.ds(base, CH)], mb.at[b], lsem).wait()
                nb = 1 - b
                nxt = base + CH

                @pl.when(i + 1 < NCH_S)
                def _():
                    pltpu.sync_copy(dst_hbm.at[pl.ds(nxt, CH)], di.at[nb])
                    pltpu.async_copy(rows_hbm.at[pl.ds(nxt, CH)], mb.at[nb], lsem)

                pltpu.sync_copy(mb.at[b], table.at[di.at[b]], add=True)
            return carry

        lax.fori_loop(0, NCH_S // 2, outer, 0)
        tbase = base0 + NCH_S * CH
        pltpu.sync_copy(dst_hbm.at[pl.ds(tbase, CH_ST)], dit)
        pltpu.sync_copy(rows_hbm.at[pl.ds(tbase, CH_ST)], mbt)
        pltpu.sync_copy(mbt, table.at[dit], add=True)

    @pl.when(cid == 0)
    def _run_msg():
        run(msg_hbm)

    @pl.when(cid == 1)
    def _run_den():
        run(exr_hbm)

    plsc.subcore_barrier()

    @pl.when(sid == 0)
    def _dump():
        pltpu.sync_copy(table, out_hbm.at[cid])


def _epi_body(slab_ref, h_ref, wa_ref, ba_ref, al_ref, g_ref, b_ref,
              wo_ref, bo_ref, o_ref):
    num = slab_ref[0]
    den = slab_ref[1]
    agg = num / (den + 1e-16)
    out = jnp.dot(agg, wa_ref[...], preferred_element_type=jnp.float32) + ba_ref[...]
    alpha = al_ref[0, 0]
    out = out * alpha + h_ref[...] * (1.0 - alpha)
    mu = jnp.mean(out, axis=1, keepdims=True)
    var = jnp.mean((out - mu) ** 2, axis=1, keepdims=True)
    out = (out - mu) / jnp.sqrt(var + 1e-5) * g_ref[...] + b_ref[...]
    o_ref[...] = jnp.dot(out, wo_ref[...], preferred_element_type=jnp.float32) + bo_ref[...]


def _phase_e(slab, h, Wa, ba, alpha, ln_g, ln_b, W_out, b_out):
    BN = 1000
    grid = (N // BN,)
    return pl.pallas_call(
        _epi_body,
        grid=grid,
        in_specs=[
            pl.BlockSpec((NC, BN, D), lambda i: (0, i, 0)),
            pl.BlockSpec((BN, D), lambda i: (i, 0)),
            pl.BlockSpec((D, D), lambda i: (0, 0)),
            pl.BlockSpec((1, D), lambda i: (0, 0)),
            pl.BlockSpec((1, 1), lambda i: (0, 0)),
            pl.BlockSpec((1, D), lambda i: (0, 0)),
            pl.BlockSpec((1, D), lambda i: (0, 0)),
            pl.BlockSpec((D, D), lambda i: (0, 0)),
            pl.BlockSpec((1, D), lambda i: (0, 0)),
        ],
        out_specs=pl.BlockSpec((BN, D), lambda i: (i, 0)),
        out_shape=jax.ShapeDtypeStruct((N, D), jnp.float32),
    )(slab, h, Wa, ba, alpha, ln_g, ln_b, W_out, b_out)


def kernel(x, edge_index, W_adapt, b_adapt, Wk, bk, Wv, bv, Wq, bq,
           Wa, ba, rel_pri, rel_att, rel_msg, skip, ln_g, ln_b,
           W_out, b_out):
    f32 = jnp.float32
    scale = jnp.repeat(rel_pri[0] / np.sqrt(DK), DK)
    Wq_eff = Wq * scale[None, :]
    bq_eff = bq * scale
    Wk_eff = jnp.einsum('dhi,hij->dhj', Wk.reshape(D, H, DK), rel_att[0]).reshape(D, D)
    bk_eff = jnp.einsum('hi,hij->hj', bk.reshape(H, DK), rel_att[0]).reshape(D)
    Wv_eff = jnp.einsum('dhi,hij->dhj', Wv.reshape(D, H, DK), rel_msg[0]).reshape(D, D)
    bv_eff = jnp.einsum('hi,hij->hj', bv.reshape(H, DK), rel_msg[0]).reshape(D)
    Wcat = jnp.concatenate([Wq_eff, Wk_eff, Wv_eff], axis=1)
    bcat = jnp.concatenate([bq_eff, bk_eff, bv_eff])[None, :]

    h, q, k, v = _phase_a(x.astype(f32), W_adapt, b_adapt[None, :], Wcat, bcat)

    src = edge_index[0].astype(jnp.int32)
    dst = edge_index[1].astype(jnp.int32)

    qd, ks, vs = _gather_kernel(q, k, v, dst, src)
    msg, exr = _phase_c(qd, ks, vs)
    slab = _scatter_kernel(msg, exr, dst, jnp.zeros((N, D), f32))

    alpha = jax.nn.sigmoid(skip[0]).reshape(1, 1)
    return _phase_e(slab, h, Wa, ba[None, :], alpha,
                    ln_g[None, :], ln_b[None, :], W_out, b_out[None, :])
